# Initial kernel scaffold; baseline (speedup 1.0000x reference)
import jax
import jax.numpy as jnp
from jax.experimental import pallas as pl


def kernel(x, edge_index_r0, edge_index_r1, edge_index_r2, W1_r0, b1_r0, W1_r1, b1_r1, W1_r2, b1_r2, W2_r0, b2_r0, W2_r1, b2_r1, W2_r2, b2_r2, Wc, bc):
    raise NotImplementedError("write your pallas kernel here")


# speedup vs baseline: 4.3331x; 4.3331x over previous
import functools

import jax
import jax.numpy as jnp
from jax import lax
from jax.experimental import pallas as pl
from jax.experimental.pallas import tpu as pltpu
from jax.experimental.pallas import tpu_sc as plsc

N = 50000
D = 128
E = 200000

NC = 2
NS = 16
LANE = 16

RANGE = 12544
TRASH = 128
NP = 4 * RANGE
EP = 200704
ET = EP // NS
CH = 1792
NCHUNK = ET // CH
CAP = ET + TRASH + LANE
DUMP = ET + TRASH
PS = NP // NS
KR = 128
ROWS_T = RANGE // NS
ZR = (RANGE + TRASH) // NS

BLK = 1024
NB = NP // BLK

_MESH = plsc.VectorSubcoreMesh(
    core_axis_name="c", subcore_axis_name="s", num_cores=NC, num_subcores=NS)


ET2 = EP // (NC * NS)


@functools.partial(
    pl.kernel,
    out_type=tuple(jax.ShapeDtypeStruct((NC * NP,), jnp.float32)
                   for _ in range(6)),
    mesh=_MESH,
    scratch_types=[
        pltpu.VMEM_SHARED((NP,), jnp.float32),
        pltpu.VMEM_SHARED((NP,), jnp.float32),
        pltpu.VMEM_SHARED((NP,), jnp.float32),
        pltpu.VMEM_SHARED((NP,), jnp.float32),
        pltpu.VMEM_SHARED((NP,), jnp.float32),
        pltpu.VMEM_SHARED((NP,), jnp.float32),
        pltpu.VMEM((ET2,), jnp.int32),
        pltpu.VMEM((ET2,), jnp.float32),
        pltpu.VMEM((PS,), jnp.float32),
        pltpu.VMEM((PS,), jnp.float32),
    ],
)
def _deg_kernel(i0, i1, i2, i3, i4, i5, ones_hbm, zvec_hbm,
                d0, d1, d2, d3, d4, d5,
                h0, h1, h2, h3, h4, h5, idxb, oneb, zb, stage):
    c = lax.axis_index("c")
    s = lax.axis_index("s")
    so = pl.multiple_of(s * PS, 8)
    eo = pl.multiple_of((c * NS + s) * ET2, 8)
    oo = pl.multiple_of(c * NP + s * PS, 8)
    idx_refs = [i0, i1, i2, i3, i4, i5]
    deg_refs = [d0, d1, d2, d3, d4, d5]
    hists = [h0, h1, h2, h3, h4, h5]
    pltpu.sync_copy(ones_hbm, oneb)
    pltpu.sync_copy(zvec_hbm, zb)
    for h in range(6):
        pltpu.sync_copy(zb, hists[h].at[pl.ds(so, PS)])
    plsc.subcore_barrier()
    for h in range(6):
        pltpu.sync_copy(idx_refs[h].at[pl.ds(eo, ET2)], idxb)
        pltpu.sync_copy(oneb, hists[h].at[idxb], add=True)
    plsc.subcore_barrier()
    for h in range(6):
        pltpu.sync_copy(hists[h].at[pl.ds(so, PS)], stage)
        pltpu.sync_copy(stage, deg_refs[h].at[pl.ds(oo, PS)])


EC = 128
NCH2 = ET // EC


@functools.partial(
    pl.kernel,
    out_type=(jax.ShapeDtypeStruct((NP, D), jnp.float32),
              jax.ShapeDtypeStruct((NC * NP,), jnp.float32)),
    mesh=_MESH,
    scratch_types=[
        pltpu.VMEM_SHARED((RANGE + TRASH, D), jnp.float32),
        pltpu.VMEM_SHARED((NP,), jnp.float32),
        pltpu.VMEM((EC,), jnp.int32),
        pltpu.VMEM((EC,), jnp.int32),
        pltpu.VMEM((EC,), jnp.int32),
        pltpu.VMEM((EC,), jnp.float32),
        pltpu.VMEM((EC, D), jnp.float32),
        pltpu.VMEM((PS,), jnp.float32),
        pltpu.SemaphoreType.DMA,
    ],
)
def _agg_kernel(xs_hbm, src_hbm, dst_hbm, cdst_hbm, zvec_hbm, zrows_hbm,
                agg_hbm, sp_hbm,
                accv, sacc, ebs, ebd, dchunk, vbuf, rowa, zb, sem):
    c = lax.axis_index("c")
    s = lax.axis_index("s")
    lo = c * (2 * RANGE)
    so = pl.multiple_of(s * PS, 8)
    lanes16 = lax.broadcasted_iota(jnp.int32, (LANE,), 0)

    pltpu.sync_copy(zvec_hbm, zb)
    pltpu.sync_copy(zb, sacc.at[pl.ds(so, PS)])
    plsc.subcore_barrier()

    for p in range(2):
        range_lo = lo + p * RANGE
        zo = pl.multiple_of(s * ZR, 8)
        pltpu.sync_copy(zrows_hbm, accv.at[pl.ds(zo, ZR)])
        plsc.subcore_barrier()

        def chunk_body(j, _):
            eb = pl.multiple_of(s * ET + j * EC, 8)
            pltpu.sync_copy(src_hbm.at[pl.ds(eb, EC)], ebs)
            pltpu.sync_copy(dst_hbm.at[pl.ds(eb, EC)], ebd)
            if p == 0:
                @pl.when(c == lax.rem(j, 2))
                def _():
                    pltpu.async_copy(cdst_hbm.at[ebd], vbuf, sem).wait()
                    pltpu.sync_copy(vbuf, sacc.at[ebs], add=True)
            for g in range(EC // LANE):
                dv = ebd[pl.ds(g * LANE, LANE)]
                dl = dv - range_lo
                m = (dl >= 0) & (dl < RANGE)
                trash_g = RANGE + (g % 8) * LANE + lanes16
                dchunk[pl.ds(g * LANE, LANE)] = jnp.where(m, dl, trash_g)
            pltpu.async_copy(xs_hbm.at[ebs], rowa, sem).wait()
            pltpu.sync_copy(rowa, accv.at[dchunk], add=True)
            return 0

        lax.fori_loop(0, NCH2, chunk_body, 0)
        plsc.subcore_barrier()
        go = pl.multiple_of(range_lo + s * ROWS_T, 8)
        pltpu.sync_copy(accv.at[pl.ds(s * ROWS_T, ROWS_T)],
                        agg_hbm.at[pl.ds(go, ROWS_T)])
        plsc.subcore_barrier()

    oo = pl.multiple_of(c * NP + s * PS, 8)
    pltpu.sync_copy(sacc.at[pl.ds(so, PS)], zb)
    pltpu.sync_copy(zb, sp_hbm.at[pl.ds(oo, PS)])


def _prep_body(x_ref, deg_ref, xs0_ref, xs1_ref, xs2_ref, cv_ref):
    degb = deg_ref[...].reshape(6, 2, BLK).sum(axis=1)
    cv = lax.rsqrt(jnp.maximum(degb, 1.0))
    cv_ref[...] = cv.reshape(6, 1, 1, BLK)
    xb = x_ref[...]
    xs0_ref[...] = xb * cv[0][:, None]
    xs1_ref[...] = xb * cv[2][:, None]
    xs2_ref[...] = xb * cv[4][:, None]


def _prep(x_pad, deg4):
    return pl.pallas_call(
        _prep_body,
        grid=(NB,),
        in_specs=[
            pl.BlockSpec((BLK, D), lambda i: (i, 0)),
            pl.BlockSpec((6, 2, 1, 1, BLK), lambda i: (0, 0, i, 0, 0)),
        ],
        out_specs=[
            pl.BlockSpec((BLK, D), lambda i: (i, 0)),
            pl.BlockSpec((BLK, D), lambda i: (i, 0)),
            pl.BlockSpec((BLK, D), lambda i: (i, 0)),
            pl.BlockSpec((6, 1, 1, BLK), lambda i: (0, i, 0, 0)),
        ],
        out_shape=[
            jax.ShapeDtypeStruct((NP, D), jnp.float32),
            jax.ShapeDtypeStruct((NP, D), jnp.float32),
            jax.ShapeDtypeStruct((NP, D), jnp.float32),
            jax.ShapeDtypeStruct((6, NB, 1, BLK), jnp.float32),
        ],
    )(x_pad, deg4)


def _dense_body(vec_ref, a0, a1, a2, w1_ref, b1_ref, out_ref):
    i = pl.program_id(0)
    vb = vec_ref[...].reshape(12, BLK)
    aggs = [a0, a1, a2]
    acc = jnp.zeros((BLK, D), jnp.float32)
    for r in range(3):
        acc = acc + jnp.dot(aggs[r][...] * vb[r][:, None], w1_ref[r],
                            preferred_element_type=jnp.float32)
    h = jnp.maximum(acc + b1_ref[0:1, :], 0.0)
    rowpos = i * BLK + lax.broadcasted_iota(jnp.int32, (BLK,), 0)
    valid = (rowpos < N).astype(jnp.float32)
    rows = [((vb[3 + r] * (vb[6 + 2 * r] + vb[7 + 2 * r])) * valid)[None]
            for r in range(3)]
    c8 = jnp.concatenate(rows + [jnp.zeros((5, BLK), jnp.float32)], axis=0)
    rblk = jnp.dot(c8, h, preferred_element_type=jnp.float32)

    @pl.when(i == 0)
    def _():
        out_ref[...] = jnp.zeros_like(out_ref)

    out_ref[...] += rblk


def _dense(vecs4, a0, a1, a2, w1s, b1sum):
    return pl.pallas_call(
        _dense_body,
        grid=(NB,),
        in_specs=[
            pl.BlockSpec((12, 1, 1, BLK), lambda i: (0, i, 0, 0)),
            pl.BlockSpec((BLK, D), lambda i: (i, 0)),
            pl.BlockSpec((BLK, D), lambda i: (i, 0)),
            pl.BlockSpec((BLK, D), lambda i: (i, 0)),
            pl.BlockSpec((3, D, D), lambda i: (0, 0, 0)),
            pl.BlockSpec((8, D), lambda i: (0, 0)),
        ],
        out_specs=pl.BlockSpec((8, D), lambda i: (0, 0)),
        out_shape=jax.ShapeDtypeStruct((8, D), jnp.float32),
    )(vecs4, a0, a1, a2, w1s, b1sum)


def _out_body(rv_ref, w2_ref, b2_ref, wc_ref, bc_ref, o_ref):
    m = jnp.dot(rv_ref[...], w2_ref[...],
                preferred_element_type=jnp.float32) * (1.0 / N) + b2_ref[...]
    o_ref[...] = jnp.dot(m, wc_ref[...],
                         preferred_element_type=jnp.float32) + bc_ref[...]


def _outk(rv, w2big, b2sum, wcp, bcp):
    return pl.pallas_call(
        _out_body,
        out_shape=jax.ShapeDtypeStruct((8, D), jnp.float32),
    )(rv, w2big, b2sum, wcp, bcp)


def kernel(x, edge_index_r0, edge_index_r1, edge_index_r2,
           W1_r0, b1_r0, W1_r1, b1_r1, W1_r2, b1_r2,
           W2_r0, b2_r0, W2_r1, b2_r1, W2_r2, b2_r2,
           Wc, bc):
    f32 = jnp.float32
    pad_idx = N + (jnp.arange(EP - E, dtype=jnp.int32) % (NP - N))
    srcs, dsts = [], []
    for e in (edge_index_r0, edge_index_r1, edge_index_r2):
        srcs.append(jnp.concatenate([e[0].astype(jnp.int32), pad_idx]))
        dsts.append(jnp.concatenate([e[1].astype(jnp.int32), pad_idx]))
    ones_e = jnp.ones((ET2,), f32)
    zvec = jnp.zeros((PS,), f32)
    zrows = jnp.zeros((ZR, D), f32)

    degs = _deg_kernel(srcs[0], dsts[0], srcs[1], dsts[1],
                       srcs[2], dsts[2], ones_e, zvec)
    deg = jnp.stack(degs)

    x_pad = jnp.concatenate([x, jnp.zeros((NP - N, D), f32)], axis=0)
    deg4 = deg.reshape(6, 2, NB, 1, BLK)
    xs0, xs1, xs2, cv4 = _prep(x_pad, deg4)
    cv = cv4.reshape(6, NP)

    xss = (xs0, xs1, xs2)
    aggs, sps = [], []
    for r in range(3):
        agg_r, sp_r = _agg_kernel(
            xss[r], srcs[r], dsts[r], cv[2 * r + 1], zvec, zrows)
        aggs.append(agg_r)
        sps.append(sp_r.reshape(2, NP))

    vecs = jnp.concatenate([cv[1::2], cv[0::2]] + sps, axis=0)
    vecs4 = vecs.reshape(12, NB, 1, BLK)
    b1sum = jnp.tile((b1_r0 + b1_r1 + b1_r2)[None], (8, 1))
    w1s = jnp.stack([W1_r0, W1_r1, W1_r2])
    racc = _dense(vecs4, aggs[0], aggs[1], aggs[2], w1s, b1sum)

    rv = jnp.pad(racc[0:3].reshape(1, 3 * D), ((0, 7), (0, 0)))
    w2big = jnp.concatenate([W2_r0, W2_r1, W2_r2], axis=0)
    b2sum = jnp.tile((b2_r0 + b2_r1 + b2_r2)[None], (8, 1))
    wcp = jnp.pad(Wc, ((0, 0), (0, D - 10)))
    bcp = jnp.tile(jnp.pad(bc, (0, D - 10))[None], (8, 1))
    out8 = _outk(rv, w2big, b2sum, wcp, bcp)
    return out8[0:1, 0:10]


# speedup vs baseline: 6.9104x; 1.5948x over previous
import functools

import jax
import jax.numpy as jnp
from jax import lax
from jax.experimental import pallas as pl
from jax.experimental.pallas import tpu as pltpu
from jax.experimental.pallas import tpu_sc as plsc

N = 50000
D = 128
E = 200000

NC = 2
NS = 16
LANE = 16

RANGE = 12544
TRASH = 128
NP = 4 * RANGE
EP = 200704
ET = EP // NS
CH = 1792
NCHUNK = ET // CH
CAP = ET + TRASH + LANE
DUMP = ET + TRASH
PS = NP // NS
KR = 128
ROWS_T = RANGE // NS
ZR = (RANGE + TRASH) // NS

BLK = 1024
NB = NP // BLK

_MESH = plsc.VectorSubcoreMesh(
    core_axis_name="c", subcore_axis_name="s", num_cores=NC, num_subcores=NS)


ET2 = EP // (NC * NS)


@functools.partial(
    pl.kernel,
    out_type=tuple(jax.ShapeDtypeStruct((NC * NP,), jnp.float32)
                   for _ in range(6)),
    mesh=_MESH,
    scratch_types=[
        pltpu.VMEM_SHARED((NP,), jnp.float32),
        pltpu.VMEM_SHARED((NP,), jnp.float32),
        pltpu.VMEM_SHARED((NP,), jnp.float32),
        pltpu.VMEM_SHARED((NP,), jnp.float32),
        pltpu.VMEM_SHARED((NP,), jnp.float32),
        pltpu.VMEM_SHARED((NP,), jnp.float32),
        pltpu.VMEM((ET2,), jnp.int32),
        pltpu.VMEM((ET2,), jnp.float32),
        pltpu.VMEM((PS,), jnp.float32),
        pltpu.VMEM((PS,), jnp.float32),
    ],
)
def _deg_kernel(i0, i1, i2, i3, i4, i5, ones_hbm, zvec_hbm,
                d0, d1, d2, d3, d4, d5,
                h0, h1, h2, h3, h4, h5, idxb, oneb, zb, stage):
    c = lax.axis_index("c")
    s = lax.axis_index("s")
    so = pl.multiple_of(s * PS, 8)
    eo = pl.multiple_of((c * NS + s) * ET2, 8)
    oo = pl.multiple_of(c * NP + s * PS, 8)
    idx_refs = [i0, i1, i2, i3, i4, i5]
    deg_refs = [d0, d1, d2, d3, d4, d5]
    hists = [h0, h1, h2, h3, h4, h5]
    pltpu.sync_copy(ones_hbm, oneb)
    pltpu.sync_copy(zvec_hbm, zb)
    for h in range(6):
        pltpu.sync_copy(zb, hists[h].at[pl.ds(so, PS)])
    plsc.subcore_barrier()
    for h in range(6):
        pltpu.sync_copy(idx_refs[h].at[pl.ds(eo, ET2)], idxb)
        pltpu.sync_copy(oneb, hists[h].at[idxb], add=True)
    plsc.subcore_barrier()
    for h in range(6):
        pltpu.sync_copy(hists[h].at[pl.ds(so, PS)], stage)
        pltpu.sync_copy(stage, deg_refs[h].at[pl.ds(oo, PS)])


EC = 64
EBK = 896
NBLK = ET // EBK
NSUB = EBK // EC


@functools.partial(
    pl.kernel,
    out_type=(jax.ShapeDtypeStruct((NP, D), jnp.float32),
              jax.ShapeDtypeStruct((NC * NP,), jnp.float32)),
    mesh=_MESH,
    scratch_types=[
        pltpu.VMEM_SHARED((RANGE + TRASH, D), jnp.float32),
        pltpu.VMEM_SHARED((NP,), jnp.float32),
        pltpu.VMEM((EBK,), jnp.int32),
        pltpu.VMEM((EBK,), jnp.int32),
        pltpu.VMEM((EC,), jnp.int32),
        pltpu.VMEM((EC,), jnp.int32),
        pltpu.VMEM((EC,), jnp.int32),
        pltpu.VMEM((EC,), jnp.int32),
        pltpu.VMEM((EBK,), jnp.float32),
        pltpu.VMEM((EC, D), jnp.float32),
        pltpu.VMEM((EC, D), jnp.float32),
        pltpu.VMEM((PS,), jnp.float32),
        pltpu.SemaphoreType.DMA,
        pltpu.SemaphoreType.DMA,
        pltpu.SemaphoreType.DMA,
        pltpu.SemaphoreType.DMA,
        pltpu.SemaphoreType.DMA,
    ],
)
def _agg_kernel(xs_hbm, src_hbm, dst_hbm, cdst_hbm, zvec_hbm, zrows_hbm,
                agg_hbm, sp_hbm,
                accv, sacc, ebs, ebd, sch0, sch1, dch0, dch1, vbuf,
                rowa0, rowa1, zb, semg0, semg1, sems0, sems1, semv):
    c = lax.axis_index("c")
    s = lax.axis_index("s")
    lo = c * (2 * RANGE)
    so = pl.multiple_of(s * PS, 8)
    lanes16 = lax.broadcasted_iota(jnp.int32, (LANE,), 0)
    schs = (sch0, sch1)
    dchs = (dch0, dch1)
    rows = (rowa0, rowa1)
    semgs = (semg0, semg1)
    semss = (sems0, sems1)

    pltpu.sync_copy(zvec_hbm, zb)
    pltpu.sync_copy(zb, sacc.at[pl.ds(so, PS)])
    plsc.subcore_barrier()

    for p in range(2):
        range_lo = lo + p * RANGE
        zo = pl.multiple_of(s * ZR, 8)
        pltpu.sync_copy(zrows_hbm, accv.at[pl.ds(zo, ZR)])
        plsc.subcore_barrier()

        def block_body(m, _):
            eb = pl.multiple_of(s * ET + m * EBK, 8)
            pltpu.sync_copy(src_hbm.at[pl.ds(eb, EBK)], ebs)
            pltpu.sync_copy(dst_hbm.at[pl.ds(eb, EBK)], ebd)
            if p == 0:
                @pl.when(c == lax.rem(m, 2))
                def _():
                    pltpu.async_copy(cdst_hbm.at[ebd], vbuf, semv).wait()
                    pltpu.sync_copy(vbuf, sacc.at[ebs], add=True)

            def build_idx(g):
                b = g % 2
                for k in range(EC // LANE):
                    off = g * EC + k * LANE
                    sv = ebs[pl.ds(off, LANE)]
                    dv = ebd[pl.ds(off, LANE)]
                    dl = dv - range_lo
                    mk = (dl >= 0) & (dl < RANGE)
                    dloc = jnp.where(mk, dl, RANGE + lanes16)
                    schs[b][pl.ds(k * LANE, LANE)] = sv
                    dchs[b][pl.ds(k * LANE, LANE)] = dloc

            gd = [None, None]
            sd = [None, None]
            for g in range(NSUB):
                b = g % 2
                if g >= 2:
                    sd[b].wait()
                build_idx(g)
                gd[b] = pltpu.async_copy(xs_hbm.at[schs[b]], rows[b],
                                         semgs[b])
                if g >= 1:
                    gd[1 - b].wait()
                    sd[1 - b] = pltpu.async_copy(
                        rows[1 - b], accv.at[dchs[1 - b]], semss[1 - b],
                        add=True)
            bl = (NSUB - 1) % 2
            gd[bl].wait()
            sd[bl] = pltpu.async_copy(rows[bl], accv.at[dchs[bl]],
                                      semss[bl], add=True)
            sd[0].wait()
            sd[1].wait()
            return 0

        lax.fori_loop(0, NBLK, block_body, 0)
        plsc.subcore_barrier()
        go = pl.multiple_of(range_lo + s * ROWS_T, 8)
        pltpu.sync_copy(accv.at[pl.ds(s * ROWS_T, ROWS_T)],
                        agg_hbm.at[pl.ds(go, ROWS_T)])
        plsc.subcore_barrier()

    oo = pl.multiple_of(c * NP + s * PS, 8)
    pltpu.sync_copy(sacc.at[pl.ds(so, PS)], zb)
    pltpu.sync_copy(zb, sp_hbm.at[pl.ds(oo, PS)])


def _prep_body(x_ref, deg_ref, xs0_ref, xs1_ref, xs2_ref, cv_ref):
    degb = deg_ref[...].reshape(6, 2, BLK).sum(axis=1)
    cv = lax.rsqrt(jnp.maximum(degb, 1.0))
    cv_ref[...] = cv.reshape(6, 1, 1, BLK)
    xb = x_ref[...]
    xs0_ref[...] = xb * cv[0][:, None]
    xs1_ref[...] = xb * cv[2][:, None]
    xs2_ref[...] = xb * cv[4][:, None]


def _prep(x_pad, deg4):
    return pl.pallas_call(
        _prep_body,
        grid=(NB,),
        in_specs=[
            pl.BlockSpec((BLK, D), lambda i: (i, 0)),
            pl.BlockSpec((6, 2, 1, 1, BLK), lambda i: (0, 0, i, 0, 0)),
        ],
        out_specs=[
            pl.BlockSpec((BLK, D), lambda i: (i, 0)),
            pl.BlockSpec((BLK, D), lambda i: (i, 0)),
            pl.BlockSpec((BLK, D), lambda i: (i, 0)),
            pl.BlockSpec((6, 1, 1, BLK), lambda i: (0, i, 0, 0)),
        ],
        out_shape=[
            jax.ShapeDtypeStruct((NP, D), jnp.float32),
            jax.ShapeDtypeStruct((NP, D), jnp.float32),
            jax.ShapeDtypeStruct((NP, D), jnp.float32),
            jax.ShapeDtypeStruct((6, NB, 1, BLK), jnp.float32),
        ],
    )(x_pad, deg4)


def _dense_body(vec_ref, a0, a1, a2, w1_ref, b1_ref, out_ref):
    i = pl.program_id(0)
    vb = vec_ref[...].reshape(12, BLK)
    aggs = [a0, a1, a2]
    acc = jnp.zeros((BLK, D), jnp.float32)
    for r in range(3):
        acc = acc + jnp.dot(aggs[r][...] * vb[r][:, None], w1_ref[r],
                            preferred_element_type=jnp.float32)
    h = jnp.maximum(acc + b1_ref[0:1, :], 0.0)
    rowpos = i * BLK + lax.broadcasted_iota(jnp.int32, (BLK,), 0)
    valid = (rowpos < N).astype(jnp.float32)
    rows = [((vb[3 + r] * (vb[6 + 2 * r] + vb[7 + 2 * r])) * valid)[None]
            for r in range(3)]
    c8 = jnp.concatenate(rows + [jnp.zeros((5, BLK), jnp.float32)], axis=0)
    rblk = jnp.dot(c8, h, preferred_element_type=jnp.float32)

    @pl.when(i == 0)
    def _():
        out_ref[...] = jnp.zeros_like(out_ref)

    out_ref[...] += rblk


def _dense(vecs4, a0, a1, a2, w1s, b1sum):
    return pl.pallas_call(
        _dense_body,
        grid=(NB,),
        in_specs=[
            pl.BlockSpec((12, 1, 1, BLK), lambda i: (0, i, 0, 0)),
            pl.BlockSpec((BLK, D), lambda i: (i, 0)),
            pl.BlockSpec((BLK, D), lambda i: (i, 0)),
            pl.BlockSpec((BLK, D), lambda i: (i, 0)),
            pl.BlockSpec((3, D, D), lambda i: (0, 0, 0)),
            pl.BlockSpec((8, D), lambda i: (0, 0)),
        ],
        out_specs=pl.BlockSpec((8, D), lambda i: (0, 0)),
        out_shape=jax.ShapeDtypeStruct((8, D), jnp.float32),
    )(vecs4, a0, a1, a2, w1s, b1sum)


def _out_body(rv_ref, w2_ref, b2_ref, wc_ref, bc_ref, o_ref):
    m = jnp.dot(rv_ref[...], w2_ref[...],
                preferred_element_type=jnp.float32) * (1.0 / N) + b2_ref[...]
    o_ref[...] = jnp.dot(m, wc_ref[...],
                         preferred_element_type=jnp.float32) + bc_ref[...]


def _outk(rv, w2big, b2sum, wcp, bcp):
    return pl.pallas_call(
        _out_body,
        out_shape=jax.ShapeDtypeStruct((8, D), jnp.float32),
    )(rv, w2big, b2sum, wcp, bcp)


def kernel(x, edge_index_r0, edge_index_r1, edge_index_r2,
           W1_r0, b1_r0, W1_r1, b1_r1, W1_r2, b1_r2,
           W2_r0, b2_r0, W2_r1, b2_r1, W2_r2, b2_r2,
           Wc, bc):
    f32 = jnp.float32
    pad_idx = N + (jnp.arange(EP - E, dtype=jnp.int32) % (NP - N))
    srcs, dsts = [], []
    for e in (edge_index_r0, edge_index_r1, edge_index_r2):
        srcs.append(jnp.concatenate([e[0].astype(jnp.int32), pad_idx]))
        dsts.append(jnp.concatenate([e[1].astype(jnp.int32), pad_idx]))
    ones_e = jnp.ones((ET2,), f32)
    zvec = jnp.zeros((PS,), f32)
    zrows = jnp.zeros((ZR, D), f32)

    degs = _deg_kernel(srcs[0], dsts[0], srcs[1], dsts[1],
                       srcs[2], dsts[2], ones_e, zvec)
    deg = jnp.stack(degs)

    x_pad = jnp.concatenate([x, jnp.zeros((NP - N, D), f32)], axis=0)
    deg4 = deg.reshape(6, 2, NB, 1, BLK)
    xs0, xs1, xs2, cv4 = _prep(x_pad, deg4)
    cv = cv4.reshape(6, NP)

    xss = (xs0, xs1, xs2)
    aggs, sps = [], []
    for r in range(3):
        agg_r, sp_r = _agg_kernel(
            xss[r], srcs[r], dsts[r], cv[2 * r + 1], zvec, zrows)
        aggs.append(agg_r)
        sps.append(sp_r.reshape(2, NP))

    vecs = jnp.concatenate([cv[1::2], cv[0::2]] + sps, axis=0)
    vecs4 = vecs.reshape(12, NB, 1, BLK)
    b1sum = jnp.tile((b1_r0 + b1_r1 + b1_r2)[None], (8, 1))
    w1s = jnp.stack([W1_r0, W1_r1, W1_r2])
    racc = _dense(vecs4, aggs[0], aggs[1], aggs[2], w1s, b1sum)

    rv = jnp.pad(racc[0:3].reshape(1, 3 * D), ((0, 7), (0, 0)))
    w2big = jnp.concatenate([W2_r0, W2_r1, W2_r2], axis=0)
    b2sum = jnp.tile((b2_r0 + b2_r1 + b2_r2)[None], (8, 1))
    wcp = jnp.pad(Wc, ((0, 0), (0, D - 10)))
    bcp = jnp.tile(jnp.pad(bc, (0, D - 10))[None], (8, 1))
    out8 = _outk(rv, w2big, b2sum, wcp, bcp)
    return out8[0:1, 0:10]


# speedup vs baseline: 6.9585x; 1.0070x over previous
import functools

import jax
import jax.numpy as jnp
from jax import lax
from jax.experimental import pallas as pl
from jax.experimental.pallas import tpu as pltpu
from jax.experimental.pallas import tpu_sc as plsc

N = 50000
D = 128
E = 200000

NC = 2
NS = 16
LANE = 16

RANGE = 12544
TRASH = 128
NP = 4 * RANGE
EP = 200704
ET = EP // NS
CH = 1792
NCHUNK = ET // CH
CAP = ET + TRASH + LANE
DUMP = ET + TRASH
PS = NP // NS
KR = 128
ROWS_T = RANGE // NS
ZR = (RANGE + TRASH) // NS

BLK = 1024
NB = NP // BLK

_MESH = plsc.VectorSubcoreMesh(
    core_axis_name="c", subcore_axis_name="s", num_cores=NC, num_subcores=NS)


ET2 = EP // (NC * NS)


@functools.partial(
    pl.kernel,
    out_type=tuple(jax.ShapeDtypeStruct((NC * NP,), jnp.float32)
                   for _ in range(6)),
    mesh=_MESH,
    scratch_types=[
        pltpu.VMEM_SHARED((NP,), jnp.float32),
        pltpu.VMEM_SHARED((NP,), jnp.float32),
        pltpu.VMEM_SHARED((NP,), jnp.float32),
        pltpu.VMEM_SHARED((NP,), jnp.float32),
        pltpu.VMEM_SHARED((NP,), jnp.float32),
        pltpu.VMEM_SHARED((NP,), jnp.float32),
        pltpu.VMEM((ET2,), jnp.int32),
        pltpu.VMEM((ET2,), jnp.float32),
        pltpu.VMEM((PS,), jnp.float32),
        pltpu.VMEM((PS,), jnp.float32),
    ],
)
def _deg_kernel(i0, i1, i2, i3, i4, i5, ones_hbm, zvec_hbm,
                d0, d1, d2, d3, d4, d5,
                h0, h1, h2, h3, h4, h5, idxb, oneb, zb, stage):
    c = lax.axis_index("c")
    s = lax.axis_index("s")
    so = pl.multiple_of(s * PS, 8)
    eo = pl.multiple_of((c * NS + s) * ET2, 8)
    oo = pl.multiple_of(c * NP + s * PS, 8)
    idx_refs = [i0, i1, i2, i3, i4, i5]
    deg_refs = [d0, d1, d2, d3, d4, d5]
    hists = [h0, h1, h2, h3, h4, h5]
    pltpu.sync_copy(ones_hbm, oneb)
    pltpu.sync_copy(zvec_hbm, zb)
    for h in range(6):
        pltpu.sync_copy(zb, hists[h].at[pl.ds(so, PS)])
    plsc.subcore_barrier()
    for h in range(6):
        pltpu.sync_copy(idx_refs[h].at[pl.ds(eo, ET2)], idxb)
        pltpu.sync_copy(oneb, hists[h].at[idxb], add=True)
    plsc.subcore_barrier()
    for h in range(6):
        pltpu.sync_copy(hists[h].at[pl.ds(so, PS)], stage)
        pltpu.sync_copy(stage, deg_refs[h].at[pl.ds(oo, PS)])


EC = 64
EBK = 896
NBLK = ET // EBK
NSUB = EBK // EC


@functools.partial(
    pl.kernel,
    out_type=tuple([jax.ShapeDtypeStruct((NP, D), jnp.float32)] * 3
                   + [jax.ShapeDtypeStruct((NC * NP,), jnp.float32)] * 3),
    mesh=_MESH,
    scratch_types=[
        pltpu.VMEM_SHARED((RANGE + TRASH, D), jnp.float32),
        pltpu.VMEM_SHARED((NP,), jnp.float32),
        pltpu.VMEM((EBK,), jnp.int32),
        pltpu.VMEM((EBK,), jnp.int32),
        pltpu.VMEM((EC,), jnp.int32),
        pltpu.VMEM((EC,), jnp.int32),
        pltpu.VMEM((EC,), jnp.int32),
        pltpu.VMEM((EC,), jnp.int32),
        pltpu.VMEM((EBK,), jnp.float32),
        pltpu.VMEM((EC, D), jnp.float32),
        pltpu.VMEM((EC, D), jnp.float32),
        pltpu.VMEM((PS,), jnp.float32),
        pltpu.SemaphoreType.DMA,
        pltpu.SemaphoreType.DMA,
        pltpu.SemaphoreType.DMA,
        pltpu.SemaphoreType.DMA,
        pltpu.SemaphoreType.DMA,
    ],
)
def _agg_kernel(xs0_hbm, xs1_hbm, xs2_hbm, src0_hbm, src1_hbm, src2_hbm,
                dst0_hbm, dst1_hbm, dst2_hbm, cd0_hbm, cd1_hbm, cd2_hbm,
                zvec_hbm, zrows_hbm,
                agg0_hbm, agg1_hbm, agg2_hbm, sp0_hbm, sp1_hbm, sp2_hbm,
                accv, sacc, ebs, ebd, sch0, sch1, dch0, dch1, vbuf,
                rowa0, rowa1, zb, semg0, semg1, sems0, sems1, semv):
    c = lax.axis_index("c")
    s = lax.axis_index("s")
    lo = c * (2 * RANGE)
    so = pl.multiple_of(s * PS, 8)
    oo = pl.multiple_of(c * NP + s * PS, 8)
    lanes16 = lax.broadcasted_iota(jnp.int32, (LANE,), 0)
    schs = (sch0, sch1)
    dchs = (dch0, dch1)
    rows = (rowa0, rowa1)
    semgs = (semg0, semg1)
    semss = (sems0, sems1)
    xss = (xs0_hbm, xs1_hbm, xs2_hbm)
    srcs = (src0_hbm, src1_hbm, src2_hbm)
    dsts = (dst0_hbm, dst1_hbm, dst2_hbm)
    cds = (cd0_hbm, cd1_hbm, cd2_hbm)
    aggs = (agg0_hbm, agg1_hbm, agg2_hbm)
    sps = (sp0_hbm, sp1_hbm, sp2_hbm)

    pltpu.sync_copy(zvec_hbm, zb)

    for r in range(3):
        xs_hbm, src_hbm, dst_hbm = xss[r], srcs[r], dsts[r]
        cdst_hbm, agg_hbm, sp_hbm = cds[r], aggs[r], sps[r]
        pltpu.sync_copy(zb, sacc.at[pl.ds(so, PS)])
        plsc.subcore_barrier()

        for p in range(2):
            range_lo = lo + p * RANGE
            zo = pl.multiple_of(s * ZR, 8)
            pltpu.sync_copy(zrows_hbm, accv.at[pl.ds(zo, ZR)])
            plsc.subcore_barrier()

            def block_body(m, _):
                eb = pl.multiple_of(s * ET + m * EBK, 8)
                pltpu.sync_copy(src_hbm.at[pl.ds(eb, EBK)], ebs)
                pltpu.sync_copy(dst_hbm.at[pl.ds(eb, EBK)], ebd)
                if p == 0:
                    @pl.when(c == lax.rem(m, 2))
                    def _():
                        pltpu.async_copy(cdst_hbm.at[ebd], vbuf,
                                         semv).wait()
                        pltpu.sync_copy(vbuf, sacc.at[ebs], add=True)

                def build_idx(g):
                    b = g % 2
                    for k in range(EC // LANE):
                        off = g * EC + k * LANE
                        sv = ebs[pl.ds(off, LANE)]
                        dv = ebd[pl.ds(off, LANE)]
                        dl = dv - range_lo
                        mk = (dl >= 0) & (dl < RANGE)
                        dloc = jnp.where(mk, dl, RANGE + lanes16)
                        schs[b][pl.ds(k * LANE, LANE)] = sv
                        dchs[b][pl.ds(k * LANE, LANE)] = dloc

                gd = [None, None]
                sd = [None, None]
                for g in range(NSUB):
                    b = g % 2
                    if g >= 2:
                        sd[b].wait()
                    build_idx(g)
                    gd[b] = pltpu.async_copy(xs_hbm.at[schs[b]], rows[b],
                                             semgs[b])
                    if g >= 1:
                        gd[1 - b].wait()
                        sd[1 - b] = pltpu.async_copy(
                            rows[1 - b], accv.at[dchs[1 - b]],
                            semss[1 - b], add=True)
                bl = (NSUB - 1) % 2
                gd[bl].wait()
                sd[bl] = pltpu.async_copy(rows[bl], accv.at[dchs[bl]],
                                          semss[bl], add=True)
                sd[0].wait()
                sd[1].wait()
                return 0

            lax.fori_loop(0, NBLK, block_body, 0)
            plsc.subcore_barrier()
            go = pl.multiple_of(range_lo + s * ROWS_T, 8)
            pltpu.sync_copy(accv.at[pl.ds(s * ROWS_T, ROWS_T)],
                            agg_hbm.at[pl.ds(go, ROWS_T)])
            plsc.subcore_barrier()

        pltpu.sync_copy(sacc.at[pl.ds(so, PS)], zb)
        pltpu.sync_copy(zb, sp_hbm.at[pl.ds(oo, PS)])
        plsc.subcore_barrier()
        if r < 2:
            pltpu.sync_copy(zvec_hbm, zb)


def _prep_body(x_ref, deg_ref, xs0_ref, xs1_ref, xs2_ref, cv_ref):
    degb = deg_ref[...].reshape(6, 2, BLK).sum(axis=1)
    cv = lax.rsqrt(jnp.maximum(degb, 1.0))
    cv_ref[...] = cv.reshape(6, 1, 1, BLK)
    xb = x_ref[...]
    xs0_ref[...] = xb * cv[0][:, None]
    xs1_ref[...] = xb * cv[2][:, None]
    xs2_ref[...] = xb * cv[4][:, None]


def _prep(x_pad, deg4):
    return pl.pallas_call(
        _prep_body,
        grid=(NB,),
        in_specs=[
            pl.BlockSpec((BLK, D), lambda i: (i, 0)),
            pl.BlockSpec((6, 2, 1, 1, BLK), lambda i: (0, 0, i, 0, 0)),
        ],
        out_specs=[
            pl.BlockSpec((BLK, D), lambda i: (i, 0)),
            pl.BlockSpec((BLK, D), lambda i: (i, 0)),
            pl.BlockSpec((BLK, D), lambda i: (i, 0)),
            pl.BlockSpec((6, 1, 1, BLK), lambda i: (0, i, 0, 0)),
        ],
        out_shape=[
            jax.ShapeDtypeStruct((NP, D), jnp.float32),
            jax.ShapeDtypeStruct((NP, D), jnp.float32),
            jax.ShapeDtypeStruct((NP, D), jnp.float32),
            jax.ShapeDtypeStruct((6, NB, 1, BLK), jnp.float32),
        ],
    )(x_pad, deg4)


def _dense_body(vec_ref, a0, a1, a2, w1_ref, b1_ref, out_ref):
    i = pl.program_id(0)
    vb = vec_ref[...].reshape(12, BLK)
    aggs = [a0, a1, a2]
    acc = jnp.zeros((BLK, D), jnp.float32)
    for r in range(3):
        acc = acc + jnp.dot(aggs[r][...] * vb[r][:, None], w1_ref[r],
                            preferred_element_type=jnp.float32)
    h = jnp.maximum(acc + b1_ref[0:1, :], 0.0)
    rowpos = i * BLK + lax.broadcasted_iota(jnp.int32, (BLK,), 0)
    valid = (rowpos < N).astype(jnp.float32)
    rows = [((vb[3 + r] * (vb[6 + 2 * r] + vb[7 + 2 * r])) * valid)[None]
            for r in range(3)]
    c8 = jnp.concatenate(rows + [jnp.zeros((5, BLK), jnp.float32)], axis=0)
    rblk = jnp.dot(c8, h, preferred_element_type=jnp.float32)

    @pl.when(i == 0)
    def _():
        out_ref[...] = jnp.zeros_like(out_ref)

    out_ref[...] += rblk


def _dense(vecs4, a0, a1, a2, w1s, b1sum):
    return pl.pallas_call(
        _dense_body,
        grid=(NB,),
        in_specs=[
            pl.BlockSpec((12, 1, 1, BLK), lambda i: (0, i, 0, 0)),
            pl.BlockSpec((BLK, D), lambda i: (i, 0)),
            pl.BlockSpec((BLK, D), lambda i: (i, 0)),
            pl.BlockSpec((BLK, D), lambda i: (i, 0)),
            pl.BlockSpec((3, D, D), lambda i: (0, 0, 0)),
            pl.BlockSpec((8, D), lambda i: (0, 0)),
        ],
        out_specs=pl.BlockSpec((8, D), lambda i: (0, 0)),
        out_shape=jax.ShapeDtypeStruct((8, D), jnp.float32),
    )(vecs4, a0, a1, a2, w1s, b1sum)


def _out_body(rv_ref, w2_ref, b2_ref, wc_ref, bc_ref, o_ref):
    m = jnp.dot(rv_ref[...], w2_ref[...],
                preferred_element_type=jnp.float32) * (1.0 / N) + b2_ref[...]
    o_ref[...] = jnp.dot(m, wc_ref[...],
                         preferred_element_type=jnp.float32) + bc_ref[...]


def _outk(rv, w2big, b2sum, wcp, bcp):
    return pl.pallas_call(
        _out_body,
        out_shape=jax.ShapeDtypeStruct((8, D), jnp.float32),
    )(rv, w2big, b2sum, wcp, bcp)


def kernel(x, edge_index_r0, edge_index_r1, edge_index_r2,
           W1_r0, b1_r0, W1_r1, b1_r1, W1_r2, b1_r2,
           W2_r0, b2_r0, W2_r1, b2_r1, W2_r2, b2_r2,
           Wc, bc):
    f32 = jnp.float32
    pad_idx = N + (jnp.arange(EP - E, dtype=jnp.int32) % (NP - N))
    srcs, dsts = [], []
    for e in (edge_index_r0, edge_index_r1, edge_index_r2):
        srcs.append(jnp.concatenate([e[0].astype(jnp.int32), pad_idx]))
        dsts.append(jnp.concatenate([e[1].astype(jnp.int32), pad_idx]))
    ones_e = jnp.ones((ET2,), f32)
    zvec = jnp.zeros((PS,), f32)
    zrows = jnp.zeros((ZR, D), f32)

    degs = _deg_kernel(srcs[0], dsts[0], srcs[1], dsts[1],
                       srcs[2], dsts[2], ones_e, zvec)
    deg = jnp.stack(degs)

    x_pad = jnp.concatenate([x, jnp.zeros((NP - N, D), f32)], axis=0)
    deg4 = deg.reshape(6, 2, NB, 1, BLK)
    xs0, xs1, xs2, cv4 = _prep(x_pad, deg4)
    cv = cv4.reshape(6, NP)

    agg0, agg1, agg2, sp0, sp1, sp2 = _agg_kernel(
        xs0, xs1, xs2, srcs[0], srcs[1], srcs[2],
        dsts[0], dsts[1], dsts[2], cv[1], cv[3], cv[5],
        zvec, zrows)
    aggs = [agg0, agg1, agg2]
    sps = [sp0.reshape(2, NP), sp1.reshape(2, NP), sp2.reshape(2, NP)]

    vecs = jnp.concatenate([cv[1::2], cv[0::2]] + sps, axis=0)
    vecs4 = vecs.reshape(12, NB, 1, BLK)
    b1sum = jnp.tile((b1_r0 + b1_r1 + b1_r2)[None], (8, 1))
    w1s = jnp.stack([W1_r0, W1_r1, W1_r2])
    racc = _dense(vecs4, aggs[0], aggs[1], aggs[2], w1s, b1sum)

    rv = jnp.pad(racc[0:3].reshape(1, 3 * D), ((0, 7), (0, 0)))
    w2big = jnp.concatenate([W2_r0, W2_r1, W2_r2], axis=0)
    b2sum = jnp.tile((b2_r0 + b2_r1 + b2_r2)[None], (8, 1))
    wcp = jnp.pad(Wc, ((0, 0), (0, D - 10)))
    bcp = jnp.tile(jnp.pad(bc, (0, D - 10))[None], (8, 1))
    out8 = _outk(rv, w2big, b2sum, wcp, bcp)
    return out8[0:1, 0:10]


# speedup vs baseline: 7.3405x; 1.0549x over previous
import functools

import jax
import jax.numpy as jnp
from jax import lax
from jax.experimental import pallas as pl
from jax.experimental.pallas import tpu as pltpu
from jax.experimental.pallas import tpu_sc as plsc

N = 50000
D = 128
E = 200000

NC = 2
NS = 16
LANE = 16

RANGE = 12544
TRASH = 128
NP = 4 * RANGE
EP = 200704
ET = EP // NS
CH = 1792
NCHUNK = ET // CH
CAP = ET + TRASH + LANE
DUMP = ET + TRASH
PS = NP // NS
KR = 128
ROWS_T = RANGE // NS
ZR = (RANGE + TRASH) // NS

BLK = 1024
NB = NP // BLK

_MESH = plsc.VectorSubcoreMesh(
    core_axis_name="c", subcore_axis_name="s", num_cores=NC, num_subcores=NS)


ET2 = EP // (NC * NS)


@functools.partial(
    pl.kernel,
    out_type=tuple(jax.ShapeDtypeStruct((NC * NP,), jnp.float32)
                   for _ in range(6)),
    mesh=_MESH,
    scratch_types=[
        pltpu.VMEM_SHARED((NP,), jnp.float32),
        pltpu.VMEM_SHARED((NP,), jnp.float32),
        pltpu.VMEM_SHARED((NP,), jnp.float32),
        pltpu.VMEM_SHARED((NP,), jnp.float32),
        pltpu.VMEM_SHARED((NP,), jnp.float32),
        pltpu.VMEM_SHARED((NP,), jnp.float32),
        pltpu.VMEM((ET2,), jnp.int32),
        pltpu.VMEM((ET2,), jnp.float32),
        pltpu.VMEM((PS,), jnp.float32),
        pltpu.VMEM((PS,), jnp.float32),
    ],
)
def _deg_kernel(i0, i1, i2, i3, i4, i5, ones_hbm, zvec_hbm,
                d0, d1, d2, d3, d4, d5,
                h0, h1, h2, h3, h4, h5, idxb, oneb, zb, stage):
    c = lax.axis_index("c")
    s = lax.axis_index("s")
    so = pl.multiple_of(s * PS, 8)
    eo = pl.multiple_of((c * NS + s) * ET2, 8)
    oo = pl.multiple_of(c * NP + s * PS, 8)
    idx_refs = [i0, i1, i2, i3, i4, i5]
    deg_refs = [d0, d1, d2, d3, d4, d5]
    hists = [h0, h1, h2, h3, h4, h5]
    pltpu.sync_copy(ones_hbm, oneb)
    pltpu.sync_copy(zvec_hbm, zb)
    for h in range(6):
        pltpu.sync_copy(zb, hists[h].at[pl.ds(so, PS)])
    plsc.subcore_barrier()
    for h in range(6):
        pltpu.sync_copy(idx_refs[h].at[pl.ds(eo, ET2)], idxb)
        pltpu.sync_copy(oneb, hists[h].at[idxb], add=True)
    plsc.subcore_barrier()
    for h in range(6):
        pltpu.sync_copy(hists[h].at[pl.ds(so, PS)], stage)
        pltpu.sync_copy(stage, deg_refs[h].at[pl.ds(oo, PS)])


EC = 64
EBK = 1792
NBLK = ET // EBK
NSUB = EBK // EC


@functools.partial(
    pl.kernel,
    out_type=tuple([jax.ShapeDtypeStruct((NP, D), jnp.float32)] * 3
                   + [jax.ShapeDtypeStruct((NC * NP,), jnp.float32)] * 3),
    mesh=_MESH,
    scratch_types=[
        pltpu.VMEM_SHARED((RANGE + TRASH, D), jnp.float32),
        pltpu.VMEM_SHARED((NP,), jnp.float32),
        pltpu.VMEM((EBK,), jnp.int32),
        pltpu.VMEM((EBK,), jnp.int32),
        pltpu.VMEM((EC,), jnp.int32),
        pltpu.VMEM((EC,), jnp.int32),
        pltpu.VMEM((EC,), jnp.int32),
        pltpu.VMEM((EC,), jnp.int32),
        pltpu.VMEM((EBK,), jnp.float32),
        pltpu.VMEM((EC, D), jnp.float32),
        pltpu.VMEM((EC, D), jnp.float32),
        pltpu.VMEM((PS,), jnp.float32),
        pltpu.SemaphoreType.DMA,
        pltpu.SemaphoreType.DMA,
        pltpu.SemaphoreType.DMA,
        pltpu.SemaphoreType.DMA,
        pltpu.SemaphoreType.DMA,
    ],
)
def _agg_kernel(xs0_hbm, xs1_hbm, xs2_hbm, src0_hbm, src1_hbm, src2_hbm,
                dst0_hbm, dst1_hbm, dst2_hbm, cd0_hbm, cd1_hbm, cd2_hbm,
                zvec_hbm, zrows_hbm,
                agg0_hbm, agg1_hbm, agg2_hbm, sp0_hbm, sp1_hbm, sp2_hbm,
                accv, sacc, ebs, ebd, sch0, sch1, dch0, dch1, vbuf,
                rowa0, rowa1, zb, semg0, semg1, sems0, sems1, semv):
    c = lax.axis_index("c")
    s = lax.axis_index("s")
    lo = c * (2 * RANGE)
    so = pl.multiple_of(s * PS, 8)
    oo = pl.multiple_of(c * NP + s * PS, 8)
    lanes16 = lax.broadcasted_iota(jnp.int32, (LANE,), 0)
    schs = (sch0, sch1)
    dchs = (dch0, dch1)
    rows = (rowa0, rowa1)
    semgs = (semg0, semg1)
    semss = (sems0, sems1)
    xss = (xs0_hbm, xs1_hbm, xs2_hbm)
    srcs = (src0_hbm, src1_hbm, src2_hbm)
    dsts = (dst0_hbm, dst1_hbm, dst2_hbm)
    cds = (cd0_hbm, cd1_hbm, cd2_hbm)
    aggs = (agg0_hbm, agg1_hbm, agg2_hbm)
    sps = (sp0_hbm, sp1_hbm, sp2_hbm)

    pltpu.sync_copy(zvec_hbm, zb)

    for r in range(3):
        xs_hbm, src_hbm, dst_hbm = xss[r], srcs[r], dsts[r]
        cdst_hbm, agg_hbm, sp_hbm = cds[r], aggs[r], sps[r]
        pltpu.sync_copy(zb, sacc.at[pl.ds(so, PS)])
        plsc.subcore_barrier()

        for p in range(2):
            range_lo = lo + p * RANGE
            zo = pl.multiple_of(s * ZR, 8)
            pltpu.sync_copy(zrows_hbm, accv.at[pl.ds(zo, ZR)])
            plsc.subcore_barrier()

            def block_body(m, _):
                eb = pl.multiple_of(s * ET + m * EBK, 8)
                pltpu.sync_copy(src_hbm.at[pl.ds(eb, EBK)], ebs)
                pltpu.sync_copy(dst_hbm.at[pl.ds(eb, EBK)], ebd)
                if p == 0:
                    @pl.when(c == lax.rem(m, 2))
                    def _():
                        pltpu.async_copy(cdst_hbm.at[ebd], vbuf,
                                         semv).wait()
                        pltpu.sync_copy(vbuf, sacc.at[ebs], add=True)

                def build_idx(g):
                    b = g % 2
                    for k in range(EC // LANE):
                        off = g * EC + k * LANE
                        sv = ebs[pl.ds(off, LANE)]
                        dv = ebd[pl.ds(off, LANE)]
                        dl = dv - range_lo
                        mk = (dl >= 0) & (dl < RANGE)
                        dloc = jnp.where(mk, dl, RANGE + lanes16)
                        schs[b][pl.ds(k * LANE, LANE)] = sv
                        dchs[b][pl.ds(k * LANE, LANE)] = dloc

                gd = [None, None]
                sd = [None, None]
                for g in range(NSUB):
                    b = g % 2
                    if g >= 2:
                        sd[b].wait()
                    build_idx(g)
                    gd[b] = pltpu.async_copy(xs_hbm.at[schs[b]], rows[b],
                                             semgs[b])
                    if g >= 1:
                        gd[1 - b].wait()
                        sd[1 - b] = pltpu.async_copy(
                            rows[1 - b], accv.at[dchs[1 - b]],
                            semss[1 - b], add=True)
                bl = (NSUB - 1) % 2
                gd[bl].wait()
                sd[bl] = pltpu.async_copy(rows[bl], accv.at[dchs[bl]],
                                          semss[bl], add=True)
                sd[0].wait()
                sd[1].wait()
                return 0

            lax.fori_loop(0, NBLK, block_body, 0)
            plsc.subcore_barrier()
            go = pl.multiple_of(range_lo + s * ROWS_T, 8)
            pltpu.sync_copy(accv.at[pl.ds(s * ROWS_T, ROWS_T)],
                            agg_hbm.at[pl.ds(go, ROWS_T)])
            plsc.subcore_barrier()

        pltpu.sync_copy(sacc.at[pl.ds(so, PS)], zb)
        pltpu.sync_copy(zb, sp_hbm.at[pl.ds(oo, PS)])
        plsc.subcore_barrier()
        if r < 2:
            pltpu.sync_copy(zvec_hbm, zb)


def _prep_body(x_ref, deg_ref, xs0_ref, xs1_ref, xs2_ref, cv_ref):
    degb = deg_ref[...].reshape(6, 2, BLK).sum(axis=1)
    cv = lax.rsqrt(jnp.maximum(degb, 1.0))
    cv_ref[...] = cv.reshape(6, 1, 1, BLK)
    xb = x_ref[...]
    xs0_ref[...] = xb * cv[0][:, None]
    xs1_ref[...] = xb * cv[2][:, None]
    xs2_ref[...] = xb * cv[4][:, None]


def _prep(x_pad, deg4):
    return pl.pallas_call(
        _prep_body,
        grid=(NB,),
        in_specs=[
            pl.BlockSpec((BLK, D), lambda i: (i, 0)),
            pl.BlockSpec((6, 2, 1, 1, BLK), lambda i: (0, 0, i, 0, 0)),
        ],
        out_specs=[
            pl.BlockSpec((BLK, D), lambda i: (i, 0)),
            pl.BlockSpec((BLK, D), lambda i: (i, 0)),
            pl.BlockSpec((BLK, D), lambda i: (i, 0)),
            pl.BlockSpec((6, 1, 1, BLK), lambda i: (0, i, 0, 0)),
        ],
        out_shape=[
            jax.ShapeDtypeStruct((NP, D), jnp.float32),
            jax.ShapeDtypeStruct((NP, D), jnp.float32),
            jax.ShapeDtypeStruct((NP, D), jnp.float32),
            jax.ShapeDtypeStruct((6, NB, 1, BLK), jnp.float32),
        ],
    )(x_pad, deg4)


def _dense_body(vec_ref, a0, a1, a2, w1_ref, b1_ref, out_ref):
    i = pl.program_id(0)
    vb = vec_ref[...].reshape(12, BLK)
    aggs = [a0, a1, a2]
    acc = jnp.zeros((BLK, D), jnp.float32)
    for r in range(3):
        acc = acc + jnp.dot(aggs[r][...] * vb[r][:, None], w1_ref[r],
                            preferred_element_type=jnp.float32)
    h = jnp.maximum(acc + b1_ref[0:1, :], 0.0)
    rowpos = i * BLK + lax.broadcasted_iota(jnp.int32, (BLK,), 0)
    valid = (rowpos < N).astype(jnp.float32)
    rows = [((vb[3 + r] * (vb[6 + 2 * r] + vb[7 + 2 * r])) * valid)[None]
            for r in range(3)]
    c8 = jnp.concatenate(rows + [jnp.zeros((5, BLK), jnp.float32)], axis=0)
    rblk = jnp.dot(c8, h, preferred_element_type=jnp.float32)

    @pl.when(i == 0)
    def _():
        out_ref[...] = jnp.zeros_like(out_ref)

    out_ref[...] += rblk


def _dense(vecs4, a0, a1, a2, w1s, b1sum):
    return pl.pallas_call(
        _dense_body,
        grid=(NB,),
        in_specs=[
            pl.BlockSpec((12, 1, 1, BLK), lambda i: (0, i, 0, 0)),
            pl.BlockSpec((BLK, D), lambda i: (i, 0)),
            pl.BlockSpec((BLK, D), lambda i: (i, 0)),
            pl.BlockSpec((BLK, D), lambda i: (i, 0)),
            pl.BlockSpec((3, D, D), lambda i: (0, 0, 0)),
            pl.BlockSpec((8, D), lambda i: (0, 0)),
        ],
        out_specs=pl.BlockSpec((8, D), lambda i: (0, 0)),
        out_shape=jax.ShapeDtypeStruct((8, D), jnp.float32),
    )(vecs4, a0, a1, a2, w1s, b1sum)


def _out_body(rv_ref, w2_ref, b2_ref, wc_ref, bc_ref, o_ref):
    m = jnp.dot(rv_ref[...], w2_ref[...],
                preferred_element_type=jnp.float32) * (1.0 / N) + b2_ref[...]
    o_ref[...] = jnp.dot(m, wc_ref[...],
                         preferred_element_type=jnp.float32) + bc_ref[...]


def _outk(rv, w2big, b2sum, wcp, bcp):
    return pl.pallas_call(
        _out_body,
        out_shape=jax.ShapeDtypeStruct((8, D), jnp.float32),
    )(rv, w2big, b2sum, wcp, bcp)


def kernel(x, edge_index_r0, edge_index_r1, edge_index_r2,
           W1_r0, b1_r0, W1_r1, b1_r1, W1_r2, b1_r2,
           W2_r0, b2_r0, W2_r1, b2_r1, W2_r2, b2_r2,
           Wc, bc):
    f32 = jnp.float32
    pad_idx = N + (jnp.arange(EP - E, dtype=jnp.int32) % (NP - N))
    srcs, dsts = [], []
    for e in (edge_index_r0, edge_index_r1, edge_index_r2):
        srcs.append(jnp.concatenate([e[0].astype(jnp.int32), pad_idx]))
        dsts.append(jnp.concatenate([e[1].astype(jnp.int32), pad_idx]))
    ones_e = jnp.ones((ET2,), f32)
    zvec = jnp.zeros((PS,), f32)
    zrows = jnp.zeros((ZR, D), f32)

    degs = _deg_kernel(srcs[0], dsts[0], srcs[1], dsts[1],
                       srcs[2], dsts[2], ones_e, zvec)
    deg = jnp.stack(degs)

    x_pad = jnp.concatenate([x, jnp.zeros((NP - N, D), f32)], axis=0)
    deg4 = deg.reshape(6, 2, NB, 1, BLK)
    xs0, xs1, xs2, cv4 = _prep(x_pad, deg4)
    cv = cv4.reshape(6, NP)

    agg0, agg1, agg2, sp0, sp1, sp2 = _agg_kernel(
        xs0, xs1, xs2, srcs[0], srcs[1], srcs[2],
        dsts[0], dsts[1], dsts[2], cv[1], cv[3], cv[5],
        zvec, zrows)
    aggs = [agg0, agg1, agg2]
    sps = [sp0.reshape(2, NP), sp1.reshape(2, NP), sp2.reshape(2, NP)]

    vecs = jnp.concatenate([cv[1::2], cv[0::2]] + sps, axis=0)
    vecs4 = vecs.reshape(12, NB, 1, BLK)
    b1sum = jnp.tile((b1_r0 + b1_r1 + b1_r2)[None], (8, 1))
    w1s = jnp.stack([W1_r0, W1_r1, W1_r2])
    racc = _dense(vecs4, aggs[0], aggs[1], aggs[2], w1s, b1sum)

    rv = jnp.pad(racc[0:3].reshape(1, 3 * D), ((0, 7), (0, 0)))
    w2big = jnp.concatenate([W2_r0, W2_r1, W2_r2], axis=0)
    b2sum = jnp.tile((b2_r0 + b2_r1 + b2_r2)[None], (8, 1))
    wcp = jnp.pad(Wc, ((0, 0), (0, D - 10)))
    bcp = jnp.tile(jnp.pad(bc, (0, D - 10))[None], (8, 1))
    out8 = _outk(rv, w2big, b2sum, wcp, bcp)
    return out8[0:1, 0:10]


# speedup vs baseline: 7.3407x; 1.0000x over previous
import functools

import jax
import jax.numpy as jnp
from jax import lax
from jax.experimental import pallas as pl
from jax.experimental.pallas import tpu as pltpu
from jax.experimental.pallas import tpu_sc as plsc

N = 50000
D = 128
E = 200000

NC = 2
NS = 16
LANE = 16

RANGE = 12544
TRASH = 128
NP = 4 * RANGE
EP = 200704
ET = EP // NS
PS = NP // NS
ROWS_T = RANGE // NS
ZR = (RANGE + TRASH) // NS

BLK = 1024
NB = NP // BLK

_MESH = plsc.VectorSubcoreMesh(
    core_axis_name="c", subcore_axis_name="s", num_cores=NC, num_subcores=NS)


ET2 = EP // (NC * NS)


@functools.partial(
    pl.kernel,
    out_type=tuple(jax.ShapeDtypeStruct((NC * NP,), jnp.float32)
                   for _ in range(6)),
    mesh=_MESH,
    scratch_types=[
        pltpu.VMEM_SHARED((NP,), jnp.float32),
        pltpu.VMEM_SHARED((NP,), jnp.float32),
        pltpu.VMEM_SHARED((NP,), jnp.float32),
        pltpu.VMEM_SHARED((NP,), jnp.float32),
        pltpu.VMEM_SHARED((NP,), jnp.float32),
        pltpu.VMEM_SHARED((NP,), jnp.float32),
        pltpu.VMEM((ET2,), jnp.int32),
        pltpu.VMEM((ET2,), jnp.float32),
        pltpu.VMEM((PS,), jnp.float32),
        pltpu.VMEM((PS,), jnp.float32),
    ],
)
def _deg_kernel(i0, i1, i2, i3, i4, i5, ones_hbm, zvec_hbm,
                d0, d1, d2, d3, d4, d5,
                h0, h1, h2, h3, h4, h5, idxb, oneb, zb, stage):
    c = lax.axis_index("c")
    s = lax.axis_index("s")
    so = pl.multiple_of(s * PS, 8)
    eo = pl.multiple_of((c * NS + s) * ET2, 8)
    oo = pl.multiple_of(c * NP + s * PS, 8)
    idx_refs = [i0, i1, i2, i3, i4, i5]
    deg_refs = [d0, d1, d2, d3, d4, d5]
    hists = [h0, h1, h2, h3, h4, h5]
    pltpu.sync_copy(ones_hbm, oneb)
    pltpu.sync_copy(zvec_hbm, zb)
    for h in range(6):
        pltpu.sync_copy(zb, hists[h].at[pl.ds(so, PS)])
    plsc.subcore_barrier()
    for h in range(6):
        pltpu.sync_copy(idx_refs[h].at[pl.ds(eo, ET2)], idxb)
        pltpu.sync_copy(oneb, hists[h].at[idxb], add=True)
    plsc.subcore_barrier()
    for h in range(6):
        pltpu.sync_copy(hists[h].at[pl.ds(so, PS)], stage)
        pltpu.sync_copy(stage, deg_refs[h].at[pl.ds(oo, PS)])


EC = 64
EBK = 1792
NBLK = ET // EBK
NSUB = EBK // EC


@functools.partial(
    pl.kernel,
    out_type=tuple([jax.ShapeDtypeStruct((NP, D), jnp.float32)] * 3
                   + [jax.ShapeDtypeStruct((NC * NP,), jnp.float32)] * 3),
    mesh=_MESH,
    scratch_types=[
        pltpu.VMEM_SHARED((RANGE + TRASH, D), jnp.float32),
        pltpu.VMEM_SHARED((NP,), jnp.float32),
        pltpu.VMEM((EBK,), jnp.int32),
        pltpu.VMEM((EBK,), jnp.int32),
        pltpu.VMEM((EC,), jnp.int32),
        pltpu.VMEM((EC,), jnp.int32),
        pltpu.VMEM((EC,), jnp.int32),
        pltpu.VMEM((EC,), jnp.int32),
        pltpu.VMEM((EBK,), jnp.float32),
        pltpu.VMEM((EC, D), jnp.float32),
        pltpu.VMEM((EC, D), jnp.float32),
        pltpu.VMEM((PS,), jnp.float32),
        pltpu.SemaphoreType.DMA,
        pltpu.SemaphoreType.DMA,
        pltpu.SemaphoreType.DMA,
        pltpu.SemaphoreType.DMA,
        pltpu.SemaphoreType.DMA,
    ],
)
def _agg_kernel(xs0_hbm, xs1_hbm, xs2_hbm, src0_hbm, src1_hbm, src2_hbm,
                dst0_hbm, dst1_hbm, dst2_hbm, cd0_hbm, cd1_hbm, cd2_hbm,
                zvec_hbm, zrows_hbm,
                agg0_hbm, agg1_hbm, agg2_hbm, sp0_hbm, sp1_hbm, sp2_hbm,
                accv, sacc, ebs, ebd, sch0, sch1, dch0, dch1, vbuf,
                rowa0, rowa1, zb, semg0, semg1, sems0, sems1, semv):
    c = lax.axis_index("c")
    s = lax.axis_index("s")
    lo = c * (2 * RANGE)
    so = pl.multiple_of(s * PS, 8)
    oo = pl.multiple_of(c * NP + s * PS, 8)
    lanes16 = lax.broadcasted_iota(jnp.int32, (LANE,), 0)
    schs = (sch0, sch1)
    dchs = (dch0, dch1)
    rows = (rowa0, rowa1)
    semgs = (semg0, semg1)
    semss = (sems0, sems1)
    xss = (xs0_hbm, xs1_hbm, xs2_hbm)
    srcs = (src0_hbm, src1_hbm, src2_hbm)
    dsts = (dst0_hbm, dst1_hbm, dst2_hbm)
    cds = (cd0_hbm, cd1_hbm, cd2_hbm)
    aggs = (agg0_hbm, agg1_hbm, agg2_hbm)
    sps = (sp0_hbm, sp1_hbm, sp2_hbm)

    pltpu.sync_copy(zvec_hbm, zb)

    for r in range(3):
        xs_hbm, src_hbm, dst_hbm = xss[r], srcs[r], dsts[r]
        cdst_hbm, agg_hbm, sp_hbm = cds[r], aggs[r], sps[r]
        pltpu.sync_copy(zb, sacc.at[pl.ds(so, PS)])
        plsc.subcore_barrier()

        for p in range(2):
            range_lo = lo + p * RANGE
            zo = pl.multiple_of(s * ZR, 8)
            pltpu.sync_copy(zrows_hbm, accv.at[pl.ds(zo, ZR)])
            plsc.subcore_barrier()

            def block_body(m, _):
                eb = pl.multiple_of(s * ET + m * EBK, 8)
                pltpu.sync_copy(src_hbm.at[pl.ds(eb, EBK)], ebs)
                pltpu.sync_copy(dst_hbm.at[pl.ds(eb, EBK)], ebd)
                if p == 0:
                    @pl.when(c == lax.rem(m, 2))
                    def _():
                        pltpu.async_copy(cdst_hbm.at[ebd], vbuf,
                                         semv).wait()
                        pltpu.sync_copy(vbuf, sacc.at[ebs], add=True)

                def build_idx(g):
                    b = g % 2
                    for k in range(EC // LANE):
                        off = g * EC + k * LANE
                        sv = ebs[pl.ds(off, LANE)]
                        dv = ebd[pl.ds(off, LANE)]
                        dl = dv - range_lo
                        mk = (dl >= 0) & (dl < RANGE)
                        dloc = jnp.where(mk, dl, RANGE + lanes16)
                        schs[b][pl.ds(k * LANE, LANE)] = sv
                        dchs[b][pl.ds(k * LANE, LANE)] = dloc

                gd = [None, None]
                sd = [None, None]
                for g in range(NSUB):
                    b = g % 2
                    if g >= 2:
                        sd[b].wait()
                    build_idx(g)
                    gd[b] = pltpu.async_copy(xs_hbm.at[schs[b]], rows[b],
                                             semgs[b])
                    if g >= 1:
                        gd[1 - b].wait()
                        sd[1 - b] = pltpu.async_copy(
                            rows[1 - b], accv.at[dchs[1 - b]],
                            semss[1 - b], add=True)
                bl = (NSUB - 1) % 2
                gd[bl].wait()
                sd[bl] = pltpu.async_copy(rows[bl], accv.at[dchs[bl]],
                                          semss[bl], add=True)
                sd[0].wait()
                sd[1].wait()
                return 0

            lax.fori_loop(0, NBLK, block_body, 0)
            plsc.subcore_barrier()
            go = pl.multiple_of(range_lo + s * ROWS_T, 8)
            pltpu.sync_copy(accv.at[pl.ds(s * ROWS_T, ROWS_T)],
                            agg_hbm.at[pl.ds(go, ROWS_T)])
            plsc.subcore_barrier()

        pltpu.sync_copy(sacc.at[pl.ds(so, PS)], zb)
        pltpu.sync_copy(zb, sp_hbm.at[pl.ds(oo, PS)])
        plsc.subcore_barrier()
        if r < 2:
            pltpu.sync_copy(zvec_hbm, zb)


def _prep_body(x_ref, deg_ref, xs0_ref, xs1_ref, xs2_ref, cv_ref):
    degb = deg_ref[...].reshape(6, 2, BLK).sum(axis=1)
    cv = lax.rsqrt(jnp.maximum(degb, 1.0))
    cv_ref[...] = cv.reshape(6, 1, 1, BLK)
    xb = x_ref[...]
    xs0_ref[...] = xb * cv[0][:, None]
    xs1_ref[...] = xb * cv[2][:, None]
    xs2_ref[...] = xb * cv[4][:, None]


def _prep(x_pad, deg4):
    return pl.pallas_call(
        _prep_body,
        grid=(NB,),
        in_specs=[
            pl.BlockSpec((BLK, D), lambda i: (i, 0)),
            pl.BlockSpec((6, 2, 1, 1, BLK), lambda i: (0, 0, i, 0, 0)),
        ],
        out_specs=[
            pl.BlockSpec((BLK, D), lambda i: (i, 0)),
            pl.BlockSpec((BLK, D), lambda i: (i, 0)),
            pl.BlockSpec((BLK, D), lambda i: (i, 0)),
            pl.BlockSpec((6, 1, 1, BLK), lambda i: (0, i, 0, 0)),
        ],
        out_shape=[
            jax.ShapeDtypeStruct((NP, D), jnp.float32),
            jax.ShapeDtypeStruct((NP, D), jnp.float32),
            jax.ShapeDtypeStruct((NP, D), jnp.float32),
            jax.ShapeDtypeStruct((6, NB, 1, BLK), jnp.float32),
        ],
    )(x_pad, deg4)


def _dense_body(vec_ref, a0, a1, a2, w1_ref, b1_ref, out_ref):
    i = pl.program_id(0)
    vb = vec_ref[...].reshape(12, BLK)
    aggs = [a0, a1, a2]
    acc = jnp.zeros((BLK, D), jnp.float32)
    for r in range(3):
        acc = acc + jnp.dot(aggs[r][...] * vb[r][:, None], w1_ref[r],
                            preferred_element_type=jnp.float32)
    h = jnp.maximum(acc + b1_ref[0:1, :], 0.0)
    rowpos = i * BLK + lax.broadcasted_iota(jnp.int32, (BLK,), 0)
    valid = (rowpos < N).astype(jnp.float32)
    rows = [((vb[3 + r] * (vb[6 + 2 * r] + vb[7 + 2 * r])) * valid)[None]
            for r in range(3)]
    c8 = jnp.concatenate(rows + [jnp.zeros((5, BLK), jnp.float32)], axis=0)
    rblk = jnp.dot(c8, h, preferred_element_type=jnp.float32)

    @pl.when(i == 0)
    def _():
        out_ref[...] = jnp.zeros_like(out_ref)

    out_ref[...] += rblk


def _dense(vecs4, a0, a1, a2, w1s, b1sum):
    return pl.pallas_call(
        _dense_body,
        grid=(NB,),
        in_specs=[
            pl.BlockSpec((12, 1, 1, BLK), lambda i: (0, i, 0, 0)),
            pl.BlockSpec((BLK, D), lambda i: (i, 0)),
            pl.BlockSpec((BLK, D), lambda i: (i, 0)),
            pl.BlockSpec((BLK, D), lambda i: (i, 0)),
            pl.BlockSpec((3, D, D), lambda i: (0, 0, 0)),
            pl.BlockSpec((8, D), lambda i: (0, 0)),
        ],
        out_specs=pl.BlockSpec((8, D), lambda i: (0, 0)),
        out_shape=jax.ShapeDtypeStruct((8, D), jnp.float32),
    )(vecs4, a0, a1, a2, w1s, b1sum)


def _out_body(rv_ref, w2_ref, b2_ref, wc_ref, bc_ref, o_ref):
    m = jnp.dot(rv_ref[...], w2_ref[...],
                preferred_element_type=jnp.float32) * (1.0 / N) + b2_ref[...]
    o_ref[...] = jnp.dot(m, wc_ref[...],
                         preferred_element_type=jnp.float32) + bc_ref[...]


def _outk(rv, w2big, b2sum, wcp, bcp):
    return pl.pallas_call(
        _out_body,
        out_shape=jax.ShapeDtypeStruct((8, D), jnp.float32),
    )(rv, w2big, b2sum, wcp, bcp)


def kernel(x, edge_index_r0, edge_index_r1, edge_index_r2,
           W1_r0, b1_r0, W1_r1, b1_r1, W1_r2, b1_r2,
           W2_r0, b2_r0, W2_r1, b2_r1, W2_r2, b2_r2,
           Wc, bc):
    f32 = jnp.float32
    pad_idx = N + (jnp.arange(EP - E, dtype=jnp.int32) % (NP - N))
    srcs, dsts = [], []
    for e in (edge_index_r0, edge_index_r1, edge_index_r2):
        srcs.append(jnp.concatenate([e[0].astype(jnp.int32), pad_idx]))
        dsts.append(jnp.concatenate([e[1].astype(jnp.int32), pad_idx]))
    ones_e = jnp.ones((ET2,), f32)
    zvec = jnp.zeros((PS,), f32)
    zrows = jnp.zeros((ZR, D), f32)

    degs = _deg_kernel(srcs[0], dsts[0], srcs[1], dsts[1],
                       srcs[2], dsts[2], ones_e, zvec)
    deg = jnp.stack(degs)

    x_pad = jnp.concatenate([x, jnp.zeros((NP - N, D), f32)], axis=0)
    deg4 = deg.reshape(6, 2, NB, 1, BLK)
    xs0, xs1, xs2, cv4 = _prep(x_pad, deg4)
    cv = cv4.reshape(6, NP)

    agg0, agg1, agg2, sp0, sp1, sp2 = _agg_kernel(
        xs0, xs1, xs2, srcs[0], srcs[1], srcs[2],
        dsts[0], dsts[1], dsts[2], cv[1], cv[3], cv[5],
        zvec, zrows)
    aggs = [agg0, agg1, agg2]
    sps = [sp0.reshape(2, NP), sp1.reshape(2, NP), sp2.reshape(2, NP)]

    vecs = jnp.concatenate([cv[1::2], cv[0::2]] + sps, axis=0)
    vecs4 = vecs.reshape(12, NB, 1, BLK)
    b1sum = jnp.tile((b1_r0 + b1_r1 + b1_r2)[None], (8, 1))
    w1s = jnp.stack([W1_r0, W1_r1, W1_r2])
    racc = _dense(vecs4, aggs[0], aggs[1], aggs[2], w1s, b1sum)

    rv = jnp.pad(racc[0:3].reshape(1, 3 * D), ((0, 7), (0, 0)))
    w2big = jnp.concatenate([W2_r0, W2_r1, W2_r2], axis=0)
    b2sum = jnp.tile((b2_r0 + b2_r1 + b2_r2)[None], (8, 1))
    wcp = jnp.pad(Wc, ((0, 0), (0, D - 10)))
    bcp = jnp.tile(jnp.pad(bc, (0, D - 10))[None], (8, 1))
    out8 = _outk(rv, w2big, b2sum, wcp, bcp)
    return out8[0:1, 0:10]
